# Initial kernel scaffold; baseline (speedup 1.0000x reference)
import jax
import jax.numpy as jnp
from jax.experimental import pallas as pl


def kernel(x, edge_index, edge_index2, W11, b11, W12, b12, W21, b21, W22, b22, W_ih, W_hh, b_ih, b_hh):
    raise NotImplementedError("write your pallas kernel here")


# speedup vs baseline: 7.4116x; 7.4116x over previous
import functools

import jax
import jax.numpy as jnp
from jax import lax
from jax.experimental import pallas as pl
from jax.experimental.pallas import tpu as pltpu
from jax.experimental.pallas import tpu_sc as plsc

N = 10000
F_IN = 128
DIM = 32
E = 320000

NPAD = 10240
NC, NS = 2, 16
NW = NC * NS
CH = 128
K = 79
EPT = K * CH
EPAD = NW * EPT
RPT = NPAD // NS
BL = 2048


def _mesh():
    return plsc.VectorSubcoreMesh(
        core_axis_name="c", subcore_axis_name="s", num_cores=NC, num_subcores=NS
    )


def _sc_deg_body(d1_ref, d2_ref, out_ref, idx_v, ones_v, zer_v, acc1, acc2):
    cid = lax.axis_index("c")
    sid = lax.axis_index("s")
    for i in range(CH // 16):
        ones_v[pl.ds(i * 16, 16)] = jnp.full((16,), 1.0, jnp.float32)
    for i in range(RPT // 16):
        zer_v[pl.ds(i * 16, 16)] = jnp.zeros((16,), jnp.float32)
    sl = pl.ds(sid * RPT, RPT)
    pltpu.sync_copy(zer_v, acc1.at[sl])
    pltpu.sync_copy(zer_v, acc2.at[sl])
    plsc.subcore_barrier()
    for d_ref, acc in ((d1_ref, acc1), (d2_ref, acc2)):
        pltpu.sync_copy(d_ref.at[cid, sid], idx_v)

        def body(j, carry, acc=acc):
            pltpu.sync_copy(ones_v, acc.at[idx_v.at[j]], add=True)
            return carry

        lax.fori_loop(0, K, body, 0)
    plsc.subcore_barrier()
    pltpu.sync_copy(acc1.at[sl], out_ref.at[0, cid, sl])
    pltpu.sync_copy(acc2.at[sl], out_ref.at[1, cid, sl])


@functools.cache
def _deg_kernel_fn():
    return pl.kernel(
        _sc_deg_body,
        out_type=jax.ShapeDtypeStruct((2, NC, NPAD), jnp.float32),
        mesh=_mesh(),
        scratch_types=[
            pltpu.VMEM((K, CH), jnp.int32),
            pltpu.VMEM((CH,), jnp.float32),
            pltpu.VMEM((RPT,), jnp.float32),
            pltpu.VMEM_SHARED((NPAD,), jnp.float32),
            pltpu.VMEM_SHARED((NPAD,), jnp.float32),
        ],
    )


GW = 128


def _sc_msg_body(g_ref, s1_ref, d1_ref, s2_ref, d2_ref, out_ref,
                 sidx, didx, rows, acc, sem):
    cid = lax.axis_index("c")
    sid = lax.axis_index("s")
    z16 = jnp.zeros((16,), jnp.float32)

    def zbody(i, carry):
        for u in range(GW // 16):
            rows[i, pl.ds(u * 16, 16)] = z16
        return carry

    def zero_own_slice():
        for r in range(RPT // CH):
            pltpu.sync_copy(rows, acc.at[pl.ds(sid * RPT + r * CH, CH)])

    lax.fori_loop(0, CH, zbody, 0)
    zero_own_slice()
    plsc.subcore_barrier()
    sl = pl.ds(sid * RPT, RPT)
    for set_i, (s_ref, d_ref) in enumerate(((s1_ref, d1_ref), (s2_ref, d2_ref))):
        pltpu.sync_copy(s_ref.at[cid, sid], sidx)
        pltpu.sync_copy(d_ref.at[cid, sid], didx)

        def body(j, carry):
            pltpu.async_copy(g_ref.at[sidx.at[j]], rows, sem).wait()
            pltpu.sync_copy(rows, acc.at[didx.at[j]], add=True)
            return carry

        lax.fori_loop(0, K, body, 0)
        plsc.subcore_barrier()
        pltpu.sync_copy(acc.at[sl], out_ref.at[set_i, cid, sl])
        if set_i == 0:
            lax.fori_loop(0, CH, zbody, 0)
            zero_own_slice()
            plsc.subcore_barrier()


@functools.cache
def _msg_kernel_fn():
    return pl.kernel(
        _sc_msg_body,
        out_type=jax.ShapeDtypeStruct((2, NC, NPAD, GW), jnp.float32),
        mesh=_mesh(),
        scratch_types=[
            pltpu.VMEM((K, CH), jnp.int32),
            pltpu.VMEM((K, CH), jnp.int32),
            pltpu.VMEM((CH, GW), jnp.float32),
            pltpu.VMEM_SHARED((NPAD, GW), jnp.float32),
            pltpu.SemaphoreType.DMA,
        ],
    )


def _pack_g(dis1, dis2, h):
    bl = h.shape[0]
    return jnp.concatenate(
        [dis1 * h[:, :DIM], dis2 * h[:, DIM:],
         jnp.zeros((bl, GW - 2 * DIM), jnp.float32)], axis=1)


def _tc_prep_body(degp_ref, xp_ref, wc_ref, g_ref, hs_ref, dis_ref):
    deg1 = degp_ref[0, 0] + degp_ref[0, 1] + 1.0
    deg2 = degp_ref[1, 0] + degp_ref[1, 1] + 1.0
    dis1 = 1.0 / jnp.sqrt(deg1)
    dis2 = 1.0 / jnp.sqrt(deg2)
    h = jnp.dot(xp_ref[...], wc_ref[...], preferred_element_type=jnp.float32)
    g_ref[...] = _pack_g(dis1, dis2, h)
    hs_ref[...] = h
    dis_ref[...] = jnp.concatenate([dis1, dis2], axis=1)


def _tc_layer_body(p_ref, hs_ref, dis_ref, b1_ref, b2_ref, wc_ref,
                   g_ref, hs2_ref):
    dis1 = dis_ref[:, 0:1]
    dis2 = dis_ref[:, 1:2]
    agg1 = p_ref[0, 0][:, :DIM] + p_ref[0, 1][:, :DIM]
    agg2 = p_ref[1, 0][:, DIM:2 * DIM] + p_ref[1, 1][:, DIM:2 * DIM]
    x1 = jnp.maximum(dis1 * agg1 + dis1 * dis1 * hs_ref[:, :DIM] + b1_ref[...], 0.0)
    x2 = jnp.maximum(dis2 * agg2 + dis2 * dis2 * hs_ref[:, DIM:] + b2_ref[...], 0.0)
    x12 = jnp.concatenate([x1, x2], axis=1)
    h2 = jnp.dot(x12, wc_ref[...], preferred_element_type=jnp.float32)
    g_ref[...] = _pack_g(dis1, dis2, h2)
    hs2_ref[...] = h2


def _tc_pre_body(q_ref, hs_ref, dis_ref, b1_ref, b2_ref, wih_ref, bih_ref,
                 bhh_ref, pre_ref):
    dis1 = dis_ref[:, 0:1]
    dis2 = dis_ref[:, 1:2]
    agg1 = q_ref[0, 0][:, :DIM] + q_ref[0, 1][:, :DIM]
    agg2 = q_ref[1, 0][:, DIM:2 * DIM] + q_ref[1, 1][:, DIM:2 * DIM]
    x1 = jnp.maximum(dis1 * agg1 + dis1 * dis1 * hs_ref[:, :DIM] + b1_ref[...], 0.0)
    x2 = jnp.maximum(dis2 * agg2 + dis2 * dis2 * hs_ref[:, DIM:] + b2_ref[...], 0.0)
    x12 = jnp.concatenate([x1, x2], axis=1)
    pre_ref[...] = (
        jnp.dot(x12, wih_ref[...], preferred_element_type=jnp.float32)
        + bih_ref[...] + bhh_ref[...]
    )


def _tc_lstm_body(pre_ref, whh_ref, ys_ref, hn_ref, cn_ref):
    whh = whh_ref[...]

    def step(t, carry):
        h, c = carry
        g = pre_ref[pl.ds(t, 1), :] + jnp.dot(
            h, whh, preferred_element_type=jnp.float32)
        gi = 1.0 / (1.0 + jnp.exp(-g[:, 0:DIM]))
        gf = 1.0 / (1.0 + jnp.exp(-g[:, DIM:2 * DIM]))
        gg = jnp.tanh(g[:, 2 * DIM:3 * DIM])
        go = 1.0 / (1.0 + jnp.exp(-g[:, 3 * DIM:]))
        c2 = gf * c + gi * gg
        h2 = go * jnp.tanh(c2)
        ys_ref[pl.ds(t, 1), :] = h2
        return (h2, c2)

    z = jnp.zeros((1, DIM), jnp.float32)
    h, c = lax.fori_loop(0, N, step, (z, z))
    hn_ref[...] = h
    cn_ref[...] = c


_GRID = NPAD // BL


def _prep_call(degp4, xp, wc):
    return pl.pallas_call(
        _tc_prep_body,
        grid=(_GRID,),
        in_specs=[
            pl.BlockSpec((2, NC, BL, 1), lambda i: (0, 0, i, 0)),
            pl.BlockSpec((BL, F_IN), lambda i: (i, 0)),
            pl.BlockSpec((F_IN, 2 * DIM), lambda i: (0, 0)),
        ],
        out_specs=[
            pl.BlockSpec((BL, GW), lambda i: (i, 0)),
            pl.BlockSpec((BL, 2 * DIM), lambda i: (i, 0)),
            pl.BlockSpec((BL, 2), lambda i: (i, 0)),
        ],
        out_shape=[
            jax.ShapeDtypeStruct((NPAD, GW), jnp.float32),
            jax.ShapeDtypeStruct((NPAD, 2 * DIM), jnp.float32),
            jax.ShapeDtypeStruct((NPAD, 2), jnp.float32),
        ],
    )(degp4, xp, wc)


def _layer_call(p, hs, dis, b1, b2, wc):
    return pl.pallas_call(
        _tc_layer_body,
        grid=(_GRID,),
        in_specs=[
            pl.BlockSpec((2, NC, BL, GW), lambda i: (0, 0, i, 0)),
            pl.BlockSpec((BL, 2 * DIM), lambda i: (i, 0)),
            pl.BlockSpec((BL, 2), lambda i: (i, 0)),
            pl.BlockSpec((1, DIM), lambda i: (0, 0)),
            pl.BlockSpec((1, DIM), lambda i: (0, 0)),
            pl.BlockSpec((2 * DIM, 2 * DIM), lambda i: (0, 0)),
        ],
        out_specs=[
            pl.BlockSpec((BL, GW), lambda i: (i, 0)),
            pl.BlockSpec((BL, 2 * DIM), lambda i: (i, 0)),
        ],
        out_shape=[
            jax.ShapeDtypeStruct((NPAD, GW), jnp.float32),
            jax.ShapeDtypeStruct((NPAD, 2 * DIM), jnp.float32),
        ],
    )(p, hs, dis, b1, b2, wc)


def _pre_call(q, hs2, dis, b1, b2, wih_t, bih, bhh):
    return pl.pallas_call(
        _tc_pre_body,
        grid=(_GRID,),
        in_specs=[
            pl.BlockSpec((2, NC, BL, GW), lambda i: (0, 0, i, 0)),
            pl.BlockSpec((BL, 2 * DIM), lambda i: (i, 0)),
            pl.BlockSpec((BL, 2), lambda i: (i, 0)),
            pl.BlockSpec((1, DIM), lambda i: (0, 0)),
            pl.BlockSpec((1, DIM), lambda i: (0, 0)),
            pl.BlockSpec((2 * DIM, 4 * DIM), lambda i: (0, 0)),
            pl.BlockSpec((1, 4 * DIM), lambda i: (0, 0)),
            pl.BlockSpec((1, 4 * DIM), lambda i: (0, 0)),
        ],
        out_specs=[pl.BlockSpec((BL, 4 * DIM), lambda i: (i, 0))],
        out_shape=[jax.ShapeDtypeStruct((NPAD, 4 * DIM), jnp.float32)],
    )(q, hs2, dis, b1, b2, wih_t, bih, bhh)[0]


def _lstm_call(pre, whh_t):
    return pl.pallas_call(
        _tc_lstm_body,
        out_shape=[
            jax.ShapeDtypeStruct((N, DIM), jnp.float32),
            jax.ShapeDtypeStruct((1, DIM), jnp.float32),
            jax.ShapeDtypeStruct((1, DIM), jnp.float32),
        ],
    )(pre, whh_t)


def _pad_edges(ei):
    src, dst = ei[0], ei[1]
    pad = EPAD - E
    fill = N + (jnp.arange(pad, dtype=jnp.int32) % (NPAD - N))
    srcp = jnp.concatenate([src, fill]).reshape(NC, NS, K, CH)
    dstp = jnp.concatenate([dst, fill]).reshape(NC, NS, K, CH)
    return srcp, dstp


def kernel(x, edge_index, edge_index2, W11, b11, W12, b12, W21, b21, W22, b22,
           W_ih, W_hh, b_ih, b_hh):
    xp = jnp.pad(x, ((0, NPAD - N), (0, 0)))
    s1, d1 = _pad_edges(edge_index)
    s2, d2 = _pad_edges(edge_index2)

    degp = _deg_kernel_fn()(d1, d2)
    degp4 = degp.reshape(2, NC, NPAD, 1)
    wc1 = jnp.concatenate([W11, W12], axis=1)
    g, hs, dis = _prep_call(degp4, xp, wc1)

    p = _msg_kernel_fn()(g, s1, d1, s2, d2)
    wc2 = jnp.concatenate([W21, W22], axis=1)
    g2nd, hs2 = _layer_call(p, hs, dis, b11.reshape(1, DIM),
                            b12.reshape(1, DIM), wc2)

    q = _msg_kernel_fn()(g2nd, s1, d1, s2, d2)
    pre = _pre_call(q, hs2, dis, b21.reshape(1, DIM), b22.reshape(1, DIM),
                    W_ih.T, b_ih.reshape(1, 4 * DIM), b_hh.reshape(1, 4 * DIM))

    ys, hn, cn = _lstm_call(pre, W_hh.T)
    return ys[None], hn[None], cn[None]


# speedup vs baseline: 8.2866x; 1.1181x over previous
import functools

import jax
import jax.numpy as jnp
from jax import lax
from jax.experimental import pallas as pl
from jax.experimental.pallas import tpu as pltpu
from jax.experimental.pallas import tpu_sc as plsc

N = 10000
F_IN = 128
DIM = 32
E = 320000

NPAD = 10240
NC, NS = 2, 16
NW = NC * NS
CH = 128
K = 79
EPT = K * CH
EPAD = NW * EPT
RPT = NPAD // NS
BL = 2048


def _mesh():
    return plsc.VectorSubcoreMesh(
        core_axis_name="c", subcore_axis_name="s", num_cores=NC, num_subcores=NS
    )


def _sc_deg_body(d1_ref, d2_ref, out_ref, idx_v, ones_v, zer_v, acc1, acc2):
    cid = lax.axis_index("c")
    sid = lax.axis_index("s")
    for i in range(CH // 16):
        ones_v[pl.ds(i * 16, 16)] = jnp.full((16,), 1.0, jnp.float32)
    for i in range(RPT // 16):
        zer_v[pl.ds(i * 16, 16)] = jnp.zeros((16,), jnp.float32)
    sl = pl.ds(sid * RPT, RPT)
    pltpu.sync_copy(zer_v, acc1.at[sl])
    pltpu.sync_copy(zer_v, acc2.at[sl])
    plsc.subcore_barrier()
    for d_ref, acc in ((d1_ref, acc1), (d2_ref, acc2)):
        pltpu.sync_copy(d_ref.at[cid, sid], idx_v)

        def body(j, carry, acc=acc):
            pltpu.sync_copy(ones_v, acc.at[idx_v.at[j]], add=True)
            return carry

        lax.fori_loop(0, K, body, 0)
    plsc.subcore_barrier()
    pltpu.sync_copy(acc1.at[sl], out_ref.at[0, cid, sl])
    pltpu.sync_copy(acc2.at[sl], out_ref.at[1, cid, sl])


@functools.cache
def _deg_kernel_fn():
    return pl.kernel(
        _sc_deg_body,
        out_type=jax.ShapeDtypeStruct((2, NC, NPAD), jnp.float32),
        mesh=_mesh(),
        scratch_types=[
            pltpu.VMEM((K, CH), jnp.int32),
            pltpu.VMEM((CH,), jnp.float32),
            pltpu.VMEM((RPT,), jnp.float32),
            pltpu.VMEM_SHARED((NPAD,), jnp.float32),
            pltpu.VMEM_SHARED((NPAD,), jnp.float32),
        ],
    )


GW = 128


def _sc_msg_body(g_ref, s1_ref, d1_ref, s2_ref, d2_ref, out_ref,
                 sidx, didx, rows, acc, sem):
    cid = lax.axis_index("c")
    sid = lax.axis_index("s")
    z16 = jnp.zeros((16,), jnp.float32)

    def zbody(i, carry):
        for u in range(GW // 16):
            rows[i, pl.ds(u * 16, 16)] = z16
        return carry

    def zero_own_slice():
        for r in range(RPT // CH):
            pltpu.sync_copy(rows, acc.at[pl.ds(sid * RPT + r * CH, CH)])

    lax.fori_loop(0, CH, zbody, 0)
    zero_own_slice()
    plsc.subcore_barrier()
    sl = pl.ds(sid * RPT, RPT)
    for set_i, (s_ref, d_ref) in enumerate(((s1_ref, d1_ref), (s2_ref, d2_ref))):
        pltpu.sync_copy(s_ref.at[cid, sid], sidx)
        pltpu.sync_copy(d_ref.at[cid, sid], didx)

        def body(j, carry):
            pltpu.async_copy(g_ref.at[sidx.at[j]], rows, sem).wait()
            pltpu.sync_copy(rows, acc.at[didx.at[j]], add=True)
            return carry

        lax.fori_loop(0, K, body, 0)
        plsc.subcore_barrier()
        pltpu.sync_copy(acc.at[sl], out_ref.at[set_i, cid, sl])
        if set_i == 0:
            lax.fori_loop(0, CH, zbody, 0)
            zero_own_slice()
            plsc.subcore_barrier()


@functools.cache
def _msg_kernel_fn():
    return pl.kernel(
        _sc_msg_body,
        out_type=jax.ShapeDtypeStruct((2, NC, NPAD, GW), jnp.float32),
        mesh=_mesh(),
        scratch_types=[
            pltpu.VMEM((K, CH), jnp.int32),
            pltpu.VMEM((K, CH), jnp.int32),
            pltpu.VMEM((CH, GW), jnp.float32),
            pltpu.VMEM_SHARED((NPAD, GW), jnp.float32),
            pltpu.SemaphoreType.DMA,
        ],
    )


def _pack_g(dis1, dis2, h):
    bl = h.shape[0]
    return jnp.concatenate(
        [dis1 * h[:, :DIM], dis2 * h[:, DIM:],
         jnp.zeros((bl, GW - 2 * DIM), jnp.float32)], axis=1)


def _tc_prep_body(degp_ref, xp_ref, wc_ref, g_ref, hs_ref, dis_ref):
    deg1 = degp_ref[0, 0] + degp_ref[0, 1] + 1.0
    deg2 = degp_ref[1, 0] + degp_ref[1, 1] + 1.0
    dis1 = 1.0 / jnp.sqrt(deg1)
    dis2 = 1.0 / jnp.sqrt(deg2)
    h = jnp.dot(xp_ref[...], wc_ref[...], preferred_element_type=jnp.float32)
    g_ref[...] = _pack_g(dis1, dis2, h)
    hs_ref[...] = h
    dis_ref[...] = jnp.concatenate([dis1, dis2], axis=1)


def _tc_layer_body(p_ref, hs_ref, dis_ref, b1_ref, b2_ref, wc_ref,
                   g_ref, hs2_ref):
    dis1 = dis_ref[:, 0:1]
    dis2 = dis_ref[:, 1:2]
    agg1 = p_ref[0, 0][:, :DIM] + p_ref[0, 1][:, :DIM]
    agg2 = p_ref[1, 0][:, DIM:2 * DIM] + p_ref[1, 1][:, DIM:2 * DIM]
    x1 = jnp.maximum(dis1 * agg1 + dis1 * dis1 * hs_ref[:, :DIM] + b1_ref[...], 0.0)
    x2 = jnp.maximum(dis2 * agg2 + dis2 * dis2 * hs_ref[:, DIM:] + b2_ref[...], 0.0)
    x12 = jnp.concatenate([x1, x2], axis=1)
    h2 = jnp.dot(x12, wc_ref[...], preferred_element_type=jnp.float32)
    g_ref[...] = _pack_g(dis1, dis2, h2)
    hs2_ref[...] = h2


def _tc_pre_body(q_ref, hs_ref, dis_ref, b1_ref, b2_ref, wih_ref, bih_ref,
                 bhh_ref, pre_ref):
    dis1 = dis_ref[:, 0:1]
    dis2 = dis_ref[:, 1:2]
    agg1 = q_ref[0, 0][:, :DIM] + q_ref[0, 1][:, :DIM]
    agg2 = q_ref[1, 0][:, DIM:2 * DIM] + q_ref[1, 1][:, DIM:2 * DIM]
    x1 = jnp.maximum(dis1 * agg1 + dis1 * dis1 * hs_ref[:, :DIM] + b1_ref[...], 0.0)
    x2 = jnp.maximum(dis2 * agg2 + dis2 * dis2 * hs_ref[:, DIM:] + b2_ref[...], 0.0)
    x12 = jnp.concatenate([x1, x2], axis=1)
    pre_ref[...] = (
        jnp.dot(x12, wih_ref[...], preferred_element_type=jnp.float32)
        + bih_ref[...] + bhh_ref[...]
    )


_UNROLL = 8


def _tc_lstm_body(pre_ref, whh_ref, ys_ref, hn_ref, cn_ref):
    whh = whh_ref[...]

    def blk(tb, carry):
        h, c = carry
        pre8 = pre_ref[pl.ds(tb * _UNROLL, _UNROLL), :]
        outs = []
        for k in range(_UNROLL):
            g = pre8[k:k + 1, :] + jnp.dot(
                h, whh, preferred_element_type=jnp.float32)
            s = 1.0 / (1.0 + jnp.exp(-g))
            tt = jnp.tanh(g)
            c = s[:, DIM:2 * DIM] * c + s[:, 0:DIM] * tt[:, 2 * DIM:3 * DIM]
            h = s[:, 3 * DIM:] * jnp.tanh(c)
            outs.append(h)
        ys_ref[pl.ds(tb * _UNROLL, _UNROLL), :] = jnp.concatenate(outs, axis=0)
        return (h, c)

    z = jnp.zeros((1, DIM), jnp.float32)
    h, c = lax.fori_loop(0, N // _UNROLL, blk, (z, z))
    hn_ref[...] = h
    cn_ref[...] = c


_GRID = NPAD // BL


def _prep_call(degp4, xp, wc):
    return pl.pallas_call(
        _tc_prep_body,
        grid=(_GRID,),
        in_specs=[
            pl.BlockSpec((2, NC, BL, 1), lambda i: (0, 0, i, 0)),
            pl.BlockSpec((BL, F_IN), lambda i: (i, 0)),
            pl.BlockSpec((F_IN, 2 * DIM), lambda i: (0, 0)),
        ],
        out_specs=[
            pl.BlockSpec((BL, GW), lambda i: (i, 0)),
            pl.BlockSpec((BL, 2 * DIM), lambda i: (i, 0)),
            pl.BlockSpec((BL, 2), lambda i: (i, 0)),
        ],
        out_shape=[
            jax.ShapeDtypeStruct((NPAD, GW), jnp.float32),
            jax.ShapeDtypeStruct((NPAD, 2 * DIM), jnp.float32),
            jax.ShapeDtypeStruct((NPAD, 2), jnp.float32),
        ],
    )(degp4, xp, wc)


def _layer_call(p, hs, dis, b1, b2, wc):
    return pl.pallas_call(
        _tc_layer_body,
        grid=(_GRID,),
        in_specs=[
            pl.BlockSpec((2, NC, BL, GW), lambda i: (0, 0, i, 0)),
            pl.BlockSpec((BL, 2 * DIM), lambda i: (i, 0)),
            pl.BlockSpec((BL, 2), lambda i: (i, 0)),
            pl.BlockSpec((1, DIM), lambda i: (0, 0)),
            pl.BlockSpec((1, DIM), lambda i: (0, 0)),
            pl.BlockSpec((2 * DIM, 2 * DIM), lambda i: (0, 0)),
        ],
        out_specs=[
            pl.BlockSpec((BL, GW), lambda i: (i, 0)),
            pl.BlockSpec((BL, 2 * DIM), lambda i: (i, 0)),
        ],
        out_shape=[
            jax.ShapeDtypeStruct((NPAD, GW), jnp.float32),
            jax.ShapeDtypeStruct((NPAD, 2 * DIM), jnp.float32),
        ],
    )(p, hs, dis, b1, b2, wc)


def _pre_call(q, hs2, dis, b1, b2, wih_t, bih, bhh):
    return pl.pallas_call(
        _tc_pre_body,
        grid=(_GRID,),
        in_specs=[
            pl.BlockSpec((2, NC, BL, GW), lambda i: (0, 0, i, 0)),
            pl.BlockSpec((BL, 2 * DIM), lambda i: (i, 0)),
            pl.BlockSpec((BL, 2), lambda i: (i, 0)),
            pl.BlockSpec((1, DIM), lambda i: (0, 0)),
            pl.BlockSpec((1, DIM), lambda i: (0, 0)),
            pl.BlockSpec((2 * DIM, 4 * DIM), lambda i: (0, 0)),
            pl.BlockSpec((1, 4 * DIM), lambda i: (0, 0)),
            pl.BlockSpec((1, 4 * DIM), lambda i: (0, 0)),
        ],
        out_specs=[pl.BlockSpec((BL, 4 * DIM), lambda i: (i, 0))],
        out_shape=[jax.ShapeDtypeStruct((NPAD, 4 * DIM), jnp.float32)],
    )(q, hs2, dis, b1, b2, wih_t, bih, bhh)[0]


def _lstm_call(pre, whh_t):
    return pl.pallas_call(
        _tc_lstm_body,
        out_shape=[
            jax.ShapeDtypeStruct((N, DIM), jnp.float32),
            jax.ShapeDtypeStruct((1, DIM), jnp.float32),
            jax.ShapeDtypeStruct((1, DIM), jnp.float32),
        ],
    )(pre, whh_t)


def _pad_edges(ei):
    src, dst = ei[0], ei[1]
    pad = EPAD - E
    fill = N + (jnp.arange(pad, dtype=jnp.int32) % (NPAD - N))
    srcp = jnp.concatenate([src, fill]).reshape(NC, NS, K, CH)
    dstp = jnp.concatenate([dst, fill]).reshape(NC, NS, K, CH)
    return srcp, dstp


def kernel(x, edge_index, edge_index2, W11, b11, W12, b12, W21, b21, W22, b22,
           W_ih, W_hh, b_ih, b_hh):
    xp = jnp.pad(x, ((0, NPAD - N), (0, 0)))
    s1, d1 = _pad_edges(edge_index)
    s2, d2 = _pad_edges(edge_index2)

    degp = _deg_kernel_fn()(d1, d2)
    degp4 = degp.reshape(2, NC, NPAD, 1)
    wc1 = jnp.concatenate([W11, W12], axis=1)
    g, hs, dis = _prep_call(degp4, xp, wc1)

    p = _msg_kernel_fn()(g, s1, d1, s2, d2)
    wc2 = jnp.concatenate([W21, W22], axis=1)
    g2nd, hs2 = _layer_call(p, hs, dis, b11.reshape(1, DIM),
                            b12.reshape(1, DIM), wc2)

    q = _msg_kernel_fn()(g2nd, s1, d1, s2, d2)
    pre = _pre_call(q, hs2, dis, b21.reshape(1, DIM), b22.reshape(1, DIM),
                    W_ih.T, b_ih.reshape(1, 4 * DIM), b_hh.reshape(1, 4 * DIM))

    ys, hn, cn = _lstm_call(pre, W_hh.T)
    return ys[None], hn[None], cn[None]


# speedup vs baseline: 18.0497x; 2.1782x over previous
import functools

import jax
import jax.numpy as jnp
from jax import lax
from jax.experimental import pallas as pl
from jax.experimental.pallas import tpu as pltpu
from jax.experimental.pallas import tpu_sc as plsc

N = 10000
F_IN = 128
DIM = 32
E = 320000

NPAD = 10240
NC, NS = 2, 16
NW = NC * NS
CH = 128
K = 79
EPT = K * CH
EPAD = NW * EPT
RPT = NPAD // NS
BL = 2048


def _mesh():
    return plsc.VectorSubcoreMesh(
        core_axis_name="c", subcore_axis_name="s", num_cores=NC, num_subcores=NS
    )


def _sc_deg_body(d1_ref, d2_ref, out_ref, idx_v, ones_v, zer_v, acc1, acc2):
    cid = lax.axis_index("c")
    sid = lax.axis_index("s")
    for i in range(CH // 16):
        ones_v[pl.ds(i * 16, 16)] = jnp.full((16,), 1.0, jnp.float32)
    for i in range(RPT // 16):
        zer_v[pl.ds(i * 16, 16)] = jnp.zeros((16,), jnp.float32)
    sl = pl.ds(sid * RPT, RPT)
    pltpu.sync_copy(zer_v, acc1.at[sl])
    pltpu.sync_copy(zer_v, acc2.at[sl])
    plsc.subcore_barrier()
    for d_ref, acc in ((d1_ref, acc1), (d2_ref, acc2)):
        pltpu.sync_copy(d_ref.at[cid, sid], idx_v)

        def body(j, carry, acc=acc):
            pltpu.sync_copy(ones_v, acc.at[idx_v.at[j]], add=True)
            return carry

        lax.fori_loop(0, K, body, 0)
    plsc.subcore_barrier()
    pltpu.sync_copy(acc1.at[sl], out_ref.at[0, cid, sl])
    pltpu.sync_copy(acc2.at[sl], out_ref.at[1, cid, sl])


@functools.cache
def _deg_kernel_fn():
    return pl.kernel(
        _sc_deg_body,
        out_type=jax.ShapeDtypeStruct((2, NC, NPAD), jnp.float32),
        mesh=_mesh(),
        scratch_types=[
            pltpu.VMEM((K, CH), jnp.int32),
            pltpu.VMEM((CH,), jnp.float32),
            pltpu.VMEM((RPT,), jnp.float32),
            pltpu.VMEM_SHARED((NPAD,), jnp.float32),
            pltpu.VMEM_SHARED((NPAD,), jnp.float32),
        ],
    )


GW = 128


def _sc_msg_body(g_ref, s1_ref, d1_ref, s2_ref, d2_ref, out_ref,
                 sidx, didx, rows, acc, sem):
    cid = lax.axis_index("c")
    sid = lax.axis_index("s")
    z16 = jnp.zeros((16,), jnp.float32)

    def zbody(i, carry):
        for u in range(GW // 16):
            rows[i, pl.ds(u * 16, 16)] = z16
        return carry

    def zero_own_slice():
        for r in range(RPT // CH):
            pltpu.sync_copy(rows, acc.at[pl.ds(sid * RPT + r * CH, CH)])

    lax.fori_loop(0, CH, zbody, 0)
    zero_own_slice()
    plsc.subcore_barrier()
    sl = pl.ds(sid * RPT, RPT)
    for set_i, (s_ref, d_ref) in enumerate(((s1_ref, d1_ref), (s2_ref, d2_ref))):
        pltpu.sync_copy(s_ref.at[cid, sid], sidx)
        pltpu.sync_copy(d_ref.at[cid, sid], didx)

        def body(j, carry):
            pltpu.async_copy(g_ref.at[sidx.at[j]], rows, sem).wait()
            pltpu.sync_copy(rows, acc.at[didx.at[j]], add=True)
            return carry

        lax.fori_loop(0, K, body, 0)
        plsc.subcore_barrier()
        pltpu.sync_copy(acc.at[sl], out_ref.at[set_i, cid, sl])
        if set_i == 0:
            lax.fori_loop(0, CH, zbody, 0)
            zero_own_slice()
            plsc.subcore_barrier()


@functools.cache
def _msg_kernel_fn():
    return pl.kernel(
        _sc_msg_body,
        out_type=jax.ShapeDtypeStruct((2, NC, NPAD, GW), jnp.float32),
        mesh=_mesh(),
        scratch_types=[
            pltpu.VMEM((K, CH), jnp.int32),
            pltpu.VMEM((K, CH), jnp.int32),
            pltpu.VMEM((CH, GW), jnp.float32),
            pltpu.VMEM_SHARED((NPAD, GW), jnp.float32),
            pltpu.SemaphoreType.DMA,
        ],
    )


def _pack_g(dis1, dis2, h):
    bl = h.shape[0]
    return jnp.concatenate(
        [dis1 * h[:, :DIM], dis2 * h[:, DIM:],
         jnp.zeros((bl, GW - 2 * DIM), jnp.float32)], axis=1)


def _tc_prep_body(degp_ref, xp_ref, wc_ref, g_ref, hs_ref, dis_ref):
    deg1 = degp_ref[0, 0] + degp_ref[0, 1] + 1.0
    deg2 = degp_ref[1, 0] + degp_ref[1, 1] + 1.0
    dis1 = 1.0 / jnp.sqrt(deg1)
    dis2 = 1.0 / jnp.sqrt(deg2)
    h = jnp.dot(xp_ref[...], wc_ref[...], preferred_element_type=jnp.float32)
    g_ref[...] = _pack_g(dis1, dis2, h)
    hs_ref[...] = h
    dis_ref[...] = jnp.concatenate([dis1, dis2], axis=1)


def _tc_layer_body(p_ref, hs_ref, dis_ref, b1_ref, b2_ref, wc_ref,
                   g_ref, hs2_ref):
    dis1 = dis_ref[:, 0:1]
    dis2 = dis_ref[:, 1:2]
    agg1 = p_ref[0, 0][:, :DIM] + p_ref[0, 1][:, :DIM]
    agg2 = p_ref[1, 0][:, DIM:2 * DIM] + p_ref[1, 1][:, DIM:2 * DIM]
    x1 = jnp.maximum(dis1 * agg1 + dis1 * dis1 * hs_ref[:, :DIM] + b1_ref[...], 0.0)
    x2 = jnp.maximum(dis2 * agg2 + dis2 * dis2 * hs_ref[:, DIM:] + b2_ref[...], 0.0)
    x12 = jnp.concatenate([x1, x2], axis=1)
    h2 = jnp.dot(x12, wc_ref[...], preferred_element_type=jnp.float32)
    g_ref[...] = _pack_g(dis1, dis2, h2)
    hs2_ref[...] = h2


def _tc_pre_body(q_ref, hs_ref, dis_ref, b1_ref, b2_ref, wih_ref, bb_ref,
                 pre_ref):
    dis1 = dis_ref[:, 0:1]
    dis2 = dis_ref[:, 1:2]
    agg1 = q_ref[0, 0][:, :DIM] + q_ref[0, 1][:, :DIM]
    agg2 = q_ref[1, 0][:, DIM:2 * DIM] + q_ref[1, 1][:, DIM:2 * DIM]
    x1 = jnp.maximum(dis1 * agg1 + dis1 * dis1 * hs_ref[:, :DIM] + b1_ref[...], 0.0)
    x2 = jnp.maximum(dis2 * agg2 + dis2 * dis2 * hs_ref[:, DIM:] + b2_ref[...], 0.0)
    x12 = jnp.concatenate([x1, x2], axis=1)
    pre_ref[...] = (
        jnp.dot(x12, wih_ref[...], preferred_element_type=jnp.float32)
        + bb_ref[...]
    )


_UNROLL = 8
LW = 4 * GW


def _tc_lstm_body(pre_ref, whh_ref, ys_ref, hn_ref, cn_ref):
    whh = whh_ref[...]

    def blk(tb, carry):
        h, c = carry
        pre8 = pre_ref[pl.ds(tb * _UNROLL, _UNROLL), :]
        outs = []
        for k in range(_UNROLL):
            g = pre8[k:k + 1, :] + jnp.dot(
                h, whh, preferred_element_type=jnp.float32)
            si = 1.0 / (1.0 + jnp.exp(-g[:, 0:GW]))
            sf = 1.0 / (1.0 + jnp.exp(-g[:, GW:2 * GW]))
            sg = jnp.tanh(g[:, 2 * GW:3 * GW])
            so = 1.0 / (1.0 + jnp.exp(-g[:, 3 * GW:]))
            c = sf * c + si * sg
            h = so * jnp.tanh(c)
            outs.append(h[:, :DIM])
        ys_ref[pl.ds(tb * _UNROLL, _UNROLL), :] = jnp.concatenate(outs, axis=0)
        return (h, c)

    z = jnp.zeros((1, GW), jnp.float32)
    h, c = lax.fori_loop(0, N // _UNROLL, blk, (z, z))
    hn_ref[...] = h[:, :DIM]
    cn_ref[...] = c[:, :DIM]


_GRID = NPAD // BL


def _prep_call(degp4, xp, wc):
    return pl.pallas_call(
        _tc_prep_body,
        grid=(_GRID,),
        in_specs=[
            pl.BlockSpec((2, NC, BL, 1), lambda i: (0, 0, i, 0)),
            pl.BlockSpec((BL, F_IN), lambda i: (i, 0)),
            pl.BlockSpec((F_IN, 2 * DIM), lambda i: (0, 0)),
        ],
        out_specs=[
            pl.BlockSpec((BL, GW), lambda i: (i, 0)),
            pl.BlockSpec((BL, 2 * DIM), lambda i: (i, 0)),
            pl.BlockSpec((BL, 2), lambda i: (i, 0)),
        ],
        out_shape=[
            jax.ShapeDtypeStruct((NPAD, GW), jnp.float32),
            jax.ShapeDtypeStruct((NPAD, 2 * DIM), jnp.float32),
            jax.ShapeDtypeStruct((NPAD, 2), jnp.float32),
        ],
    )(degp4, xp, wc)


def _layer_call(p, hs, dis, b1, b2, wc):
    return pl.pallas_call(
        _tc_layer_body,
        grid=(_GRID,),
        in_specs=[
            pl.BlockSpec((2, NC, BL, GW), lambda i: (0, 0, i, 0)),
            pl.BlockSpec((BL, 2 * DIM), lambda i: (i, 0)),
            pl.BlockSpec((BL, 2), lambda i: (i, 0)),
            pl.BlockSpec((1, DIM), lambda i: (0, 0)),
            pl.BlockSpec((1, DIM), lambda i: (0, 0)),
            pl.BlockSpec((2 * DIM, 2 * DIM), lambda i: (0, 0)),
        ],
        out_specs=[
            pl.BlockSpec((BL, GW), lambda i: (i, 0)),
            pl.BlockSpec((BL, 2 * DIM), lambda i: (i, 0)),
        ],
        out_shape=[
            jax.ShapeDtypeStruct((NPAD, GW), jnp.float32),
            jax.ShapeDtypeStruct((NPAD, 2 * DIM), jnp.float32),
        ],
    )(p, hs, dis, b1, b2, wc)


def _pre_call(q, hs2, dis, b1, b2, wih_big, bb):
    return pl.pallas_call(
        _tc_pre_body,
        grid=(_GRID,),
        in_specs=[
            pl.BlockSpec((2, NC, BL, GW), lambda i: (0, 0, i, 0)),
            pl.BlockSpec((BL, 2 * DIM), lambda i: (i, 0)),
            pl.BlockSpec((BL, 2), lambda i: (i, 0)),
            pl.BlockSpec((1, DIM), lambda i: (0, 0)),
            pl.BlockSpec((1, DIM), lambda i: (0, 0)),
            pl.BlockSpec((2 * DIM, LW), lambda i: (0, 0)),
            pl.BlockSpec((1, LW), lambda i: (0, 0)),
        ],
        out_specs=[pl.BlockSpec((BL, LW), lambda i: (i, 0))],
        out_shape=[jax.ShapeDtypeStruct((NPAD, LW), jnp.float32)],
    )(q, hs2, dis, b1, b2, wih_big, bb)[0]


def _lstm_call(pre, whh_big):
    return pl.pallas_call(
        _tc_lstm_body,
        out_shape=[
            jax.ShapeDtypeStruct((N, DIM), jnp.float32),
            jax.ShapeDtypeStruct((1, DIM), jnp.float32),
            jax.ShapeDtypeStruct((1, DIM), jnp.float32),
        ],
    )(pre, whh_big)


def _tile_gates(w):
    return jnp.concatenate(
        [jnp.tile(w[:, g * DIM:(g + 1) * DIM], (1, 4)) for g in range(4)],
        axis=1)


def _pad_edges(ei):
    src, dst = ei[0], ei[1]
    pad = EPAD - E
    fill = N + (jnp.arange(pad, dtype=jnp.int32) % (NPAD - N))
    srcp = jnp.concatenate([src, fill]).reshape(NC, NS, K, CH)
    dstp = jnp.concatenate([dst, fill]).reshape(NC, NS, K, CH)
    return srcp, dstp


def kernel(x, edge_index, edge_index2, W11, b11, W12, b12, W21, b21, W22, b22,
           W_ih, W_hh, b_ih, b_hh):
    xp = jnp.pad(x, ((0, NPAD - N), (0, 0)))
    s1, d1 = _pad_edges(edge_index)
    s2, d2 = _pad_edges(edge_index2)

    degp = _deg_kernel_fn()(d1, d2)
    degp4 = degp.reshape(2, NC, NPAD, 1)
    wc1 = jnp.concatenate([W11, W12], axis=1)
    g, hs, dis = _prep_call(degp4, xp, wc1)

    p = _msg_kernel_fn()(g, s1, d1, s2, d2)
    wc2 = jnp.concatenate([W21, W22], axis=1)
    g2nd, hs2 = _layer_call(p, hs, dis, b11.reshape(1, DIM),
                            b12.reshape(1, DIM), wc2)

    q = _msg_kernel_fn()(g2nd, s1, d1, s2, d2)
    wih_big = _tile_gates(W_ih.T)
    whh_big = jnp.tile(_tile_gates(W_hh.T) * 0.25, (4, 1))
    bb = _tile_gates((b_ih + b_hh).reshape(1, 4 * DIM))

    pre = _pre_call(q, hs2, dis, b21.reshape(1, DIM), b22.reshape(1, DIM),
                    wih_big, bb)
    ys, hn, cn = _lstm_call(pre, whh_big)
    return ys[None], hn[None], cn[None]


# speedup vs baseline: 18.1797x; 1.0072x over previous
import functools

import jax
import jax.numpy as jnp
from jax import lax
from jax.experimental import pallas as pl
from jax.experimental.pallas import tpu as pltpu
from jax.experimental.pallas import tpu_sc as plsc

N = 10000
F_IN = 128
DIM = 32
E = 320000

NPAD = 10240
NC, NS = 2, 16
NW = NC * NS
CH = 128
K = 80
EPT = K * CH
EPAD = NW * EPT
RPT = NPAD // NS
BL = 2048


def _mesh():
    return plsc.VectorSubcoreMesh(
        core_axis_name="c", subcore_axis_name="s", num_cores=NC, num_subcores=NS
    )


def _sc_deg_body(d1_ref, d2_ref, out_ref, idx_v, ones_v, zer_v, acc1, acc2):
    cid = lax.axis_index("c")
    sid = lax.axis_index("s")
    for i in range(CH // 16):
        ones_v[pl.ds(i * 16, 16)] = jnp.full((16,), 1.0, jnp.float32)
    for i in range(RPT // 16):
        zer_v[pl.ds(i * 16, 16)] = jnp.zeros((16,), jnp.float32)
    sl = pl.ds(sid * RPT, RPT)
    pltpu.sync_copy(zer_v, acc1.at[sl])
    pltpu.sync_copy(zer_v, acc2.at[sl])
    plsc.subcore_barrier()
    for d_ref, acc in ((d1_ref, acc1), (d2_ref, acc2)):
        pltpu.sync_copy(d_ref.at[cid, sid], idx_v)

        def body(j, carry, acc=acc):
            pltpu.sync_copy(ones_v, acc.at[idx_v.at[j]], add=True)
            return carry

        lax.fori_loop(0, K, body, 0)
    plsc.subcore_barrier()
    pltpu.sync_copy(acc1.at[sl], out_ref.at[0, cid, sl])
    pltpu.sync_copy(acc2.at[sl], out_ref.at[1, cid, sl])


@functools.cache
def _deg_kernel_fn():
    return pl.kernel(
        _sc_deg_body,
        out_type=jax.ShapeDtypeStruct((2, NC, NPAD), jnp.float32),
        mesh=_mesh(),
        scratch_types=[
            pltpu.VMEM((K, CH), jnp.int32),
            pltpu.VMEM((CH,), jnp.float32),
            pltpu.VMEM((RPT,), jnp.float32),
            pltpu.VMEM_SHARED((NPAD,), jnp.float32),
            pltpu.VMEM_SHARED((NPAD,), jnp.float32),
        ],
    )


GW = 128


def _sc_msg_body(g_ref, s1_ref, d1_ref, s2_ref, d2_ref, out_ref,
                 sidx, didx, rows, acc, sem):
    cid = lax.axis_index("c")
    sid = lax.axis_index("s")
    z16 = jnp.zeros((16,), jnp.float32)

    def zbody(i, carry):
        for u in range(GW // 16):
            rows[i, pl.ds(u * 16, 16)] = z16
        return carry

    lax.fori_loop(0, CH, zbody, 0)
    for r in range(RPT // CH):
        pltpu.sync_copy(rows, acc.at[pl.ds(sid * RPT + r * CH, CH)])
    plsc.subcore_barrier()
    sl = pl.ds(sid * RPT, RPT)
    for set_i, (s_ref, d_ref) in enumerate(((s1_ref, d1_ref), (s2_ref, d2_ref))):
        pltpu.sync_copy(s_ref.at[cid, sid], sidx)
        pltpu.sync_copy(d_ref.at[cid, sid], didx)

        def body(j, carry):
            pltpu.async_copy(g_ref.at[sidx.at[j]], rows, sem).wait()
            pltpu.sync_copy(rows, acc.at[didx.at[j]], add=True)
            return carry

        lax.fori_loop(0, K, body, 0)
        plsc.subcore_barrier()
        pltpu.sync_copy(acc.at[sl], out_ref.at[set_i, cid, sl])
        plsc.subcore_barrier()


@functools.cache
def _msg_kernel_fn():
    return pl.kernel(
        _sc_msg_body,
        out_type=jax.ShapeDtypeStruct((2, NC, NPAD, GW), jnp.float32),
        mesh=_mesh(),
        scratch_types=[
            pltpu.VMEM((K, CH), jnp.int32),
            pltpu.VMEM((K, CH), jnp.int32),
            pltpu.VMEM((CH, GW), jnp.float32),
            pltpu.VMEM_SHARED((NPAD, GW), jnp.float32),
            pltpu.SemaphoreType.DMA,
        ],
    )


def _pack_g(dis1, dis2, h):
    bl = h.shape[0]
    return jnp.concatenate(
        [dis1 * h[:, :DIM], dis2 * h[:, DIM:],
         jnp.zeros((bl, GW - 2 * DIM), jnp.float32)], axis=1)


def _tc_prep_body(degp_ref, xp_ref, wc_ref, g_ref, hs_ref, dis_ref):
    deg1 = degp_ref[0, 0] + degp_ref[0, 1] + 1.0
    deg2 = degp_ref[1, 0] + degp_ref[1, 1] + 1.0
    dis1 = 1.0 / jnp.sqrt(deg1)
    dis2 = 1.0 / jnp.sqrt(deg2)
    h = jnp.dot(xp_ref[...], wc_ref[...], preferred_element_type=jnp.float32)
    g_ref[...] = _pack_g(dis1, dis2, h)
    hs_ref[...] = h
    dis_ref[...] = jnp.concatenate([dis1, dis2], axis=1)


def _tc_layer_body(p_ref, hs_ref, dis_ref, b1_ref, b2_ref, wc_ref,
                   g_ref, hs2_ref):
    dis1 = dis_ref[:, 0:1]
    dis2 = dis_ref[:, 1:2]
    agg1 = p_ref[0, 0][:, :DIM] + p_ref[0, 1][:, :DIM]
    agg2 = (p_ref[1, 0][:, DIM:2 * DIM] + p_ref[1, 1][:, DIM:2 * DIM]
            - p_ref[0, 0][:, DIM:2 * DIM] - p_ref[0, 1][:, DIM:2 * DIM])
    x1 = jnp.maximum(dis1 * agg1 + dis1 * dis1 * hs_ref[:, :DIM] + b1_ref[...], 0.0)
    x2 = jnp.maximum(dis2 * agg2 + dis2 * dis2 * hs_ref[:, DIM:] + b2_ref[...], 0.0)
    x12 = jnp.concatenate([x1, x2], axis=1)
    h2 = jnp.dot(x12, wc_ref[...], preferred_element_type=jnp.float32)
    g_ref[...] = _pack_g(dis1, dis2, h2)
    hs2_ref[...] = h2


def _tc_pre_body(q_ref, hs_ref, dis_ref, b1_ref, b2_ref, wih_ref, bb_ref,
                 pre_ref):
    dis1 = dis_ref[:, 0:1]
    dis2 = dis_ref[:, 1:2]
    agg1 = q_ref[0, 0][:, :DIM] + q_ref[0, 1][:, :DIM]
    agg2 = (q_ref[1, 0][:, DIM:2 * DIM] + q_ref[1, 1][:, DIM:2 * DIM]
            - q_ref[0, 0][:, DIM:2 * DIM] - q_ref[0, 1][:, DIM:2 * DIM])
    x1 = jnp.maximum(dis1 * agg1 + dis1 * dis1 * hs_ref[:, :DIM] + b1_ref[...], 0.0)
    x2 = jnp.maximum(dis2 * agg2 + dis2 * dis2 * hs_ref[:, DIM:] + b2_ref[...], 0.0)
    x12 = jnp.concatenate([x1, x2], axis=1)
    pre_ref[...] = (
        jnp.dot(x12, wih_ref[...], preferred_element_type=jnp.float32)
        + bb_ref[...]
    )


_UNROLL = 8
LW = 4 * GW


def _tc_lstm_body(pre_ref, whh_ref, ys_ref, hn_ref, cn_ref):
    whh = whh_ref[...]

    def blk(tb, carry):
        h, c = carry
        pre8 = pre_ref[pl.ds(tb * _UNROLL, _UNROLL), :]
        outs = []
        for k in range(_UNROLL):
            g = pre8[k:k + 1, :] + jnp.dot(
                h[:, :DIM], whh, preferred_element_type=jnp.float32)
            si = 1.0 / (1.0 + jnp.exp(-g[:, 0:GW]))
            sf = 1.0 / (1.0 + jnp.exp(-g[:, GW:2 * GW]))
            sg = jnp.tanh(g[:, 2 * GW:3 * GW])
            so = 1.0 / (1.0 + jnp.exp(-g[:, 3 * GW:]))
            c = sf * c + si * sg
            h = so * jnp.tanh(c)
            outs.append(h[:, :DIM])
        ys_ref[pl.ds(tb * _UNROLL, _UNROLL), :] = jnp.concatenate(outs, axis=0)
        return (h, c)

    z = jnp.zeros((1, GW), jnp.float32)
    h, c = lax.fori_loop(0, N // _UNROLL, blk, (z, z))
    hn_ref[...] = h[:, :DIM]
    cn_ref[...] = c[:, :DIM]


_GRID = NPAD // BL


def _prep_call(degp4, xp, wc):
    return pl.pallas_call(
        _tc_prep_body,
        grid=(_GRID,),
        in_specs=[
            pl.BlockSpec((2, NC, BL, 1), lambda i: (0, 0, i, 0)),
            pl.BlockSpec((BL, F_IN), lambda i: (i, 0)),
            pl.BlockSpec((F_IN, 2 * DIM), lambda i: (0, 0)),
        ],
        out_specs=[
            pl.BlockSpec((BL, GW), lambda i: (i, 0)),
            pl.BlockSpec((BL, 2 * DIM), lambda i: (i, 0)),
            pl.BlockSpec((BL, 2), lambda i: (i, 0)),
        ],
        out_shape=[
            jax.ShapeDtypeStruct((NPAD, GW), jnp.float32),
            jax.ShapeDtypeStruct((NPAD, 2 * DIM), jnp.float32),
            jax.ShapeDtypeStruct((NPAD, 2), jnp.float32),
        ],
    )(degp4, xp, wc)


def _layer_call(p, hs, dis, b1, b2, wc):
    return pl.pallas_call(
        _tc_layer_body,
        grid=(_GRID,),
        in_specs=[
            pl.BlockSpec((2, NC, BL, GW), lambda i: (0, 0, i, 0)),
            pl.BlockSpec((BL, 2 * DIM), lambda i: (i, 0)),
            pl.BlockSpec((BL, 2), lambda i: (i, 0)),
            pl.BlockSpec((1, DIM), lambda i: (0, 0)),
            pl.BlockSpec((1, DIM), lambda i: (0, 0)),
            pl.BlockSpec((2 * DIM, 2 * DIM), lambda i: (0, 0)),
        ],
        out_specs=[
            pl.BlockSpec((BL, GW), lambda i: (i, 0)),
            pl.BlockSpec((BL, 2 * DIM), lambda i: (i, 0)),
        ],
        out_shape=[
            jax.ShapeDtypeStruct((NPAD, GW), jnp.float32),
            jax.ShapeDtypeStruct((NPAD, 2 * DIM), jnp.float32),
        ],
    )(p, hs, dis, b1, b2, wc)


def _pre_call(q, hs2, dis, b1, b2, wih_big, bb):
    return pl.pallas_call(
        _tc_pre_body,
        grid=(_GRID,),
        in_specs=[
            pl.BlockSpec((2, NC, BL, GW), lambda i: (0, 0, i, 0)),
            pl.BlockSpec((BL, 2 * DIM), lambda i: (i, 0)),
            pl.BlockSpec((BL, 2), lambda i: (i, 0)),
            pl.BlockSpec((1, DIM), lambda i: (0, 0)),
            pl.BlockSpec((1, DIM), lambda i: (0, 0)),
            pl.BlockSpec((2 * DIM, LW), lambda i: (0, 0)),
            pl.BlockSpec((1, LW), lambda i: (0, 0)),
        ],
        out_specs=[pl.BlockSpec((BL, LW), lambda i: (i, 0))],
        out_shape=[jax.ShapeDtypeStruct((NPAD, LW), jnp.float32)],
    )(q, hs2, dis, b1, b2, wih_big, bb)[0]


def _lstm_call(pre, whh_big):
    return pl.pallas_call(
        _tc_lstm_body,
        out_shape=[
            jax.ShapeDtypeStruct((N, DIM), jnp.float32),
            jax.ShapeDtypeStruct((1, DIM), jnp.float32),
            jax.ShapeDtypeStruct((1, DIM), jnp.float32),
        ],
    )(pre, whh_big)


def _tile_gates(w):
    return jnp.concatenate(
        [jnp.tile(w[:, g * DIM:(g + 1) * DIM], (1, 4)) for g in range(4)],
        axis=1)


def _pad_edges(ei):
    src, dst = ei[0], ei[1]
    pad = EPAD - E
    fill = N + (jnp.arange(pad, dtype=jnp.int32) % (NPAD - N))
    srcp = jnp.concatenate([src, fill]).reshape(NC, NS, K, CH)
    dstp = jnp.concatenate([dst, fill]).reshape(NC, NS, K, CH)
    return srcp, dstp


def kernel(x, edge_index, edge_index2, W11, b11, W12, b12, W21, b21, W22, b22,
           W_ih, W_hh, b_ih, b_hh):
    xp = jnp.pad(x, ((0, NPAD - N), (0, 0)))
    s1, d1 = _pad_edges(edge_index)
    s2, d2 = _pad_edges(edge_index2)

    degp = _deg_kernel_fn()(d1, d2)
    degp4 = degp.reshape(2, NC, NPAD, 1)
    wc1 = jnp.concatenate([W11, W12], axis=1)
    g, hs, dis = _prep_call(degp4, xp, wc1)

    p = _msg_kernel_fn()(g, s1, d1, s2, d2)
    wc2 = jnp.concatenate([W21, W22], axis=1)
    g2nd, hs2 = _layer_call(p, hs, dis, b11.reshape(1, DIM),
                            b12.reshape(1, DIM), wc2)

    q = _msg_kernel_fn()(g2nd, s1, d1, s2, d2)
    wih_big = _tile_gates(W_ih.T)
    whh_big = _tile_gates(W_hh.T)
    bb = _tile_gates((b_ih + b_hh).reshape(1, 4 * DIM))

    pre = _pre_call(q, hs2, dis, b21.reshape(1, DIM), b22.reshape(1, DIM),
                    wih_big, bb)
    ys, hn, cn = _lstm_call(pre, whh_big)
    return ys[None], hn[None], cn[None]


# speedup vs baseline: 18.2214x; 1.0023x over previous
import functools

import jax
import jax.numpy as jnp
from jax import lax
from jax.experimental import pallas as pl
from jax.experimental.pallas import tpu as pltpu
from jax.experimental.pallas import tpu_sc as plsc

N = 10000
F_IN = 128
DIM = 32
E = 320000

NPAD = 10240
NC, NS = 2, 16
NW = NC * NS
CH = 128
K = 80
EPT = K * CH
EPAD = NW * EPT
RPT = NPAD // NS
BL = 2048


def _mesh():
    return plsc.VectorSubcoreMesh(
        core_axis_name="c", subcore_axis_name="s", num_cores=NC, num_subcores=NS
    )


def _sc_deg_body(d1_ref, d2_ref, out_ref, idx_v, ones_v, zer_v, acc1, acc2):
    cid = lax.axis_index("c")
    sid = lax.axis_index("s")
    for i in range(CH // 16):
        ones_v[pl.ds(i * 16, 16)] = jnp.full((16,), 1.0, jnp.float32)
    for i in range(RPT // 16):
        zer_v[pl.ds(i * 16, 16)] = jnp.zeros((16,), jnp.float32)
    sl = pl.ds(sid * RPT, RPT)
    pltpu.sync_copy(zer_v, acc1.at[sl])
    pltpu.sync_copy(zer_v, acc2.at[sl])
    plsc.subcore_barrier()
    for d_ref, acc in ((d1_ref, acc1), (d2_ref, acc2)):
        pltpu.sync_copy(d_ref.at[cid, sid], idx_v)

        def body(j, carry, acc=acc):
            pltpu.sync_copy(ones_v, acc.at[idx_v.at[j]], add=True)
            return carry

        lax.fori_loop(0, K, body, 0)
    plsc.subcore_barrier()
    pltpu.sync_copy(acc1.at[sl], out_ref.at[0, cid, sl])
    pltpu.sync_copy(acc2.at[sl], out_ref.at[1, cid, sl])


@functools.cache
def _deg_kernel_fn():
    return pl.kernel(
        _sc_deg_body,
        out_type=jax.ShapeDtypeStruct((2, NC, NPAD), jnp.float32),
        mesh=_mesh(),
        scratch_types=[
            pltpu.VMEM((K, CH), jnp.int32),
            pltpu.VMEM((CH,), jnp.float32),
            pltpu.VMEM((RPT,), jnp.float32),
            pltpu.VMEM_SHARED((NPAD,), jnp.float32),
            pltpu.VMEM_SHARED((NPAD,), jnp.float32),
        ],
    )


GW = 128


def _sc_msg_body(g_ref, s1_ref, d1_ref, s2_ref, d2_ref, out_ref,
                 sidx, didx, rows, acc, sem):
    cid = lax.axis_index("c")
    sid = lax.axis_index("s")
    z16 = jnp.zeros((16,), jnp.float32)

    def zbody(i, carry):
        for u in range(GW // 16):
            rows[i, pl.ds(u * 16, 16)] = z16
        return carry

    lax.fori_loop(0, CH, zbody, 0)
    for r in range(RPT // CH):
        pltpu.sync_copy(rows, acc.at[pl.ds(sid * RPT + r * CH, CH)])
    plsc.subcore_barrier()
    sl = pl.ds(sid * RPT, RPT)
    for set_i, (s_ref, d_ref) in enumerate(((s1_ref, d1_ref), (s2_ref, d2_ref))):
        pltpu.sync_copy(s_ref.at[cid, sid], sidx)
        pltpu.sync_copy(d_ref.at[cid, sid], didx)

        def body(j, carry):
            pltpu.async_copy(g_ref.at[sidx.at[j]], rows, sem).wait()
            pltpu.sync_copy(rows, acc.at[didx.at[j]], add=True)
            return carry

        lax.fori_loop(0, K, body, 0)
        plsc.subcore_barrier()
        pltpu.sync_copy(acc.at[sl], out_ref.at[set_i, cid, sl])
        plsc.subcore_barrier()


@functools.cache
def _msg_kernel_fn():
    return pl.kernel(
        _sc_msg_body,
        out_type=jax.ShapeDtypeStruct((2, NC, NPAD, GW), jnp.float32),
        mesh=_mesh(),
        scratch_types=[
            pltpu.VMEM((K, CH), jnp.int32),
            pltpu.VMEM((K, CH), jnp.int32),
            pltpu.VMEM((CH, GW), jnp.float32),
            pltpu.VMEM_SHARED((NPAD, GW), jnp.float32),
            pltpu.SemaphoreType.DMA,
        ],
    )


def _pack_g(dis1, dis2, h):
    bl = h.shape[0]
    return jnp.concatenate(
        [dis1 * h[:, :DIM], dis2 * h[:, DIM:],
         jnp.zeros((bl, GW - 2 * DIM), jnp.float32)], axis=1)


def _tc_prep_body(degp_ref, xp_ref, wc_ref, g_ref, hs_ref, dis_ref):
    deg1 = degp_ref[0, 0] + degp_ref[0, 1] + 1.0
    deg2 = degp_ref[1, 0] + degp_ref[1, 1] + 1.0
    dis1 = 1.0 / jnp.sqrt(deg1)
    dis2 = 1.0 / jnp.sqrt(deg2)
    h = jnp.dot(xp_ref[...], wc_ref[...], preferred_element_type=jnp.float32)
    g_ref[...] = _pack_g(dis1, dis2, h)
    hs_ref[...] = h
    dis_ref[...] = jnp.concatenate([dis1, dis2], axis=1)


def _tc_layer_body(p_ref, hs_ref, dis_ref, b1_ref, b2_ref, wc_ref,
                   g_ref, hs2_ref):
    dis1 = dis_ref[:, 0:1]
    dis2 = dis_ref[:, 1:2]
    agg1 = p_ref[0, 0][:, :DIM] + p_ref[0, 1][:, :DIM]
    agg2 = (p_ref[1, 0][:, DIM:2 * DIM] + p_ref[1, 1][:, DIM:2 * DIM]
            - p_ref[0, 0][:, DIM:2 * DIM] - p_ref[0, 1][:, DIM:2 * DIM])
    x1 = jnp.maximum(dis1 * agg1 + dis1 * dis1 * hs_ref[:, :DIM] + b1_ref[...], 0.0)
    x2 = jnp.maximum(dis2 * agg2 + dis2 * dis2 * hs_ref[:, DIM:] + b2_ref[...], 0.0)
    x12 = jnp.concatenate([x1, x2], axis=1)
    h2 = jnp.dot(x12, wc_ref[...], preferred_element_type=jnp.float32)
    g_ref[...] = _pack_g(dis1, dis2, h2)
    hs2_ref[...] = h2


def _tc_pre_body(q_ref, hs_ref, dis_ref, b1_ref, b2_ref, wih_ref, bb_ref,
                 pre_ref):
    dis1 = dis_ref[:, 0:1]
    dis2 = dis_ref[:, 1:2]
    agg1 = q_ref[0, 0][:, :DIM] + q_ref[0, 1][:, :DIM]
    agg2 = (q_ref[1, 0][:, DIM:2 * DIM] + q_ref[1, 1][:, DIM:2 * DIM]
            - q_ref[0, 0][:, DIM:2 * DIM] - q_ref[0, 1][:, DIM:2 * DIM])
    x1 = jnp.maximum(dis1 * agg1 + dis1 * dis1 * hs_ref[:, :DIM] + b1_ref[...], 0.0)
    x2 = jnp.maximum(dis2 * agg2 + dis2 * dis2 * hs_ref[:, DIM:] + b2_ref[...], 0.0)
    x12 = jnp.concatenate([x1, x2], axis=1)
    pre_ref[...] = (
        jnp.dot(x12, wih_ref[...], preferred_element_type=jnp.float32)
        + bb_ref[...]
    )


_UNROLL = 16
LW = 4 * GW


def _tc_lstm_body(pre_ref, whh_ref, ys_ref, hn_ref, cn_ref):
    whh = whh_ref[...]

    def blk(tb, carry):
        h, c = carry
        pre8 = pre_ref[pl.ds(tb * _UNROLL, _UNROLL), :]
        outs = []
        for k in range(_UNROLL):
            g = pre8[k:k + 1, :] + jnp.dot(
                h[:, :DIM], whh, preferred_element_type=jnp.float32)
            si = 1.0 / (1.0 + jnp.exp(-g[:, 0:GW]))
            sf = 1.0 / (1.0 + jnp.exp(-g[:, GW:2 * GW]))
            sg = jnp.tanh(g[:, 2 * GW:3 * GW])
            so = 1.0 / (1.0 + jnp.exp(-g[:, 3 * GW:]))
            c = sf * c + si * sg
            h = so * jnp.tanh(c)
            outs.append(h[:, :DIM])
        ys_ref[pl.ds(tb * _UNROLL, _UNROLL), :] = jnp.concatenate(outs, axis=0)
        return (h, c)

    z = jnp.zeros((1, GW), jnp.float32)
    h, c = lax.fori_loop(0, N // _UNROLL, blk, (z, z))
    hn_ref[...] = h[:, :DIM]
    cn_ref[...] = c[:, :DIM]


_GRID = NPAD // BL


def _prep_call(degp4, xp, wc):
    return pl.pallas_call(
        _tc_prep_body,
        grid=(_GRID,),
        in_specs=[
            pl.BlockSpec((2, NC, BL, 1), lambda i: (0, 0, i, 0)),
            pl.BlockSpec((BL, F_IN), lambda i: (i, 0)),
            pl.BlockSpec((F_IN, 2 * DIM), lambda i: (0, 0)),
        ],
        out_specs=[
            pl.BlockSpec((BL, GW), lambda i: (i, 0)),
            pl.BlockSpec((BL, 2 * DIM), lambda i: (i, 0)),
            pl.BlockSpec((BL, 2), lambda i: (i, 0)),
        ],
        out_shape=[
            jax.ShapeDtypeStruct((NPAD, GW), jnp.float32),
            jax.ShapeDtypeStruct((NPAD, 2 * DIM), jnp.float32),
            jax.ShapeDtypeStruct((NPAD, 2), jnp.float32),
        ],
    )(degp4, xp, wc)


def _layer_call(p, hs, dis, b1, b2, wc):
    return pl.pallas_call(
        _tc_layer_body,
        grid=(_GRID,),
        in_specs=[
            pl.BlockSpec((2, NC, BL, GW), lambda i: (0, 0, i, 0)),
            pl.BlockSpec((BL, 2 * DIM), lambda i: (i, 0)),
            pl.BlockSpec((BL, 2), lambda i: (i, 0)),
            pl.BlockSpec((1, DIM), lambda i: (0, 0)),
            pl.BlockSpec((1, DIM), lambda i: (0, 0)),
            pl.BlockSpec((2 * DIM, 2 * DIM), lambda i: (0, 0)),
        ],
        out_specs=[
            pl.BlockSpec((BL, GW), lambda i: (i, 0)),
            pl.BlockSpec((BL, 2 * DIM), lambda i: (i, 0)),
        ],
        out_shape=[
            jax.ShapeDtypeStruct((NPAD, GW), jnp.float32),
            jax.ShapeDtypeStruct((NPAD, 2 * DIM), jnp.float32),
        ],
    )(p, hs, dis, b1, b2, wc)


def _pre_call(q, hs2, dis, b1, b2, wih_big, bb):
    return pl.pallas_call(
        _tc_pre_body,
        grid=(_GRID,),
        in_specs=[
            pl.BlockSpec((2, NC, BL, GW), lambda i: (0, 0, i, 0)),
            pl.BlockSpec((BL, 2 * DIM), lambda i: (i, 0)),
            pl.BlockSpec((BL, 2), lambda i: (i, 0)),
            pl.BlockSpec((1, DIM), lambda i: (0, 0)),
            pl.BlockSpec((1, DIM), lambda i: (0, 0)),
            pl.BlockSpec((2 * DIM, LW), lambda i: (0, 0)),
            pl.BlockSpec((1, LW), lambda i: (0, 0)),
        ],
        out_specs=[pl.BlockSpec((BL, LW), lambda i: (i, 0))],
        out_shape=[jax.ShapeDtypeStruct((NPAD, LW), jnp.float32)],
    )(q, hs2, dis, b1, b2, wih_big, bb)[0]


def _lstm_call(pre, whh_big):
    return pl.pallas_call(
        _tc_lstm_body,
        out_shape=[
            jax.ShapeDtypeStruct((N, DIM), jnp.float32),
            jax.ShapeDtypeStruct((1, DIM), jnp.float32),
            jax.ShapeDtypeStruct((1, DIM), jnp.float32),
        ],
    )(pre, whh_big)


def _tile_gates(w):
    return jnp.concatenate(
        [jnp.tile(w[:, g * DIM:(g + 1) * DIM], (1, 4)) for g in range(4)],
        axis=1)


def _pad_edges(ei):
    src, dst = ei[0], ei[1]
    pad = EPAD - E
    fill = N + (jnp.arange(pad, dtype=jnp.int32) % (NPAD - N))
    srcp = jnp.concatenate([src, fill]).reshape(NC, NS, K, CH)
    dstp = jnp.concatenate([dst, fill]).reshape(NC, NS, K, CH)
    return srcp, dstp


def kernel(x, edge_index, edge_index2, W11, b11, W12, b12, W21, b21, W22, b22,
           W_ih, W_hh, b_ih, b_hh):
    xp = jnp.pad(x, ((0, NPAD - N), (0, 0)))
    s1, d1 = _pad_edges(edge_index)
    s2, d2 = _pad_edges(edge_index2)

    degp = _deg_kernel_fn()(d1, d2)
    degp4 = degp.reshape(2, NC, NPAD, 1)
    wc1 = jnp.concatenate([W11, W12], axis=1)
    g, hs, dis = _prep_call(degp4, xp, wc1)

    p = _msg_kernel_fn()(g, s1, d1, s2, d2)
    wc2 = jnp.concatenate([W21, W22], axis=1)
    g2nd, hs2 = _layer_call(p, hs, dis, b11.reshape(1, DIM),
                            b12.reshape(1, DIM), wc2)

    q = _msg_kernel_fn()(g2nd, s1, d1, s2, d2)
    wih_big = _tile_gates(W_ih.T)
    whh_big = _tile_gates(W_hh.T)
    bb = _tile_gates((b_ih + b_hh).reshape(1, 4 * DIM))

    pre = _pre_call(q, hs2, dis, b21.reshape(1, DIM), b22.reshape(1, DIM),
                    wih_big, bb)
    ys, hn, cn = _lstm_call(pre, whh_big)
    return ys[None], hn[None], cn[None]


# speedup vs baseline: 18.2869x; 1.0036x over previous
import functools

import jax
import jax.numpy as jnp
from jax import lax
from jax.experimental import pallas as pl
from jax.experimental.pallas import tpu as pltpu
from jax.experimental.pallas import tpu_sc as plsc

N = 10000
F_IN = 128
DIM = 32
E = 320000

NPAD = 10240
NC, NS = 2, 16
NW = NC * NS
CH = 128
K = 79
EPT = K * CH
EPAD = NW * EPT
RPT = NPAD // NS
BL = 2048


def _mesh():
    return plsc.VectorSubcoreMesh(
        core_axis_name="c", subcore_axis_name="s", num_cores=NC, num_subcores=NS
    )


def _sc_deg_body(d1_ref, d2_ref, out_ref, idx_v, ones_v, zer_v, acc1, acc2):
    cid = lax.axis_index("c")
    sid = lax.axis_index("s")
    for i in range(CH // 16):
        ones_v[pl.ds(i * 16, 16)] = jnp.full((16,), 1.0, jnp.float32)
    for i in range(RPT // 16):
        zer_v[pl.ds(i * 16, 16)] = jnp.zeros((16,), jnp.float32)
    sl = pl.ds(sid * RPT, RPT)
    pltpu.sync_copy(zer_v, acc1.at[sl])
    pltpu.sync_copy(zer_v, acc2.at[sl])
    plsc.subcore_barrier()
    for d_ref, acc in ((d1_ref, acc1), (d2_ref, acc2)):
        pltpu.sync_copy(d_ref.at[cid, sid], idx_v)

        def body(j, carry, acc=acc):
            pltpu.sync_copy(ones_v, acc.at[idx_v.at[j]], add=True)
            return carry

        lax.fori_loop(0, K, body, 0)
    plsc.subcore_barrier()
    pltpu.sync_copy(acc1.at[sl], out_ref.at[0, cid, sl])
    pltpu.sync_copy(acc2.at[sl], out_ref.at[1, cid, sl])


@functools.cache
def _deg_kernel_fn():
    return pl.kernel(
        _sc_deg_body,
        out_type=jax.ShapeDtypeStruct((2, NC, NPAD), jnp.float32),
        mesh=_mesh(),
        scratch_types=[
            pltpu.VMEM((K, CH), jnp.int32),
            pltpu.VMEM((CH,), jnp.float32),
            pltpu.VMEM((RPT,), jnp.float32),
            pltpu.VMEM_SHARED((NPAD,), jnp.float32),
            pltpu.VMEM_SHARED((NPAD,), jnp.float32),
        ],
    )


GW = 128


def _sc_msg_body(g_ref, s1_ref, d1_ref, s2_ref, d2_ref, out_ref,
                 sidx, didx, rows, acc, sem):
    cid = lax.axis_index("c")
    sid = lax.axis_index("s")
    z16 = jnp.zeros((16,), jnp.float32)

    def zbody(i, carry):
        for u in range(GW // 16):
            rows[i, pl.ds(u * 16, 16)] = z16
        return carry

    lax.fori_loop(0, CH, zbody, 0)
    for r in range(RPT // CH):
        pltpu.sync_copy(rows, acc.at[pl.ds(sid * RPT + r * CH, CH)])
    plsc.subcore_barrier()
    sl = pl.ds(sid * RPT, RPT)
    for set_i, (s_ref, d_ref) in enumerate(((s1_ref, d1_ref), (s2_ref, d2_ref))):
        pltpu.sync_copy(s_ref.at[cid, sid], sidx)
        pltpu.sync_copy(d_ref.at[cid, sid], didx)

        def body(j, carry):
            pltpu.async_copy(g_ref.at[sidx.at[j]], rows, sem).wait()
            pltpu.sync_copy(rows, acc.at[didx.at[j]], add=True)
            return carry

        lax.fori_loop(0, K, body, 0)
        plsc.subcore_barrier()
        pltpu.sync_copy(acc.at[sl], out_ref.at[set_i, cid, sl])
        plsc.subcore_barrier()


@functools.cache
def _msg_kernel_fn():
    return pl.kernel(
        _sc_msg_body,
        out_type=jax.ShapeDtypeStruct((2, NC, NPAD, GW), jnp.float32),
        mesh=_mesh(),
        scratch_types=[
            pltpu.VMEM((K, CH), jnp.int32),
            pltpu.VMEM((K, CH), jnp.int32),
            pltpu.VMEM((CH, GW), jnp.float32),
            pltpu.VMEM_SHARED((NPAD, GW), jnp.float32),
            pltpu.SemaphoreType.DMA,
        ],
    )


def _pack_g(dis1, dis2, h):
    bl = h.shape[0]
    return jnp.concatenate(
        [dis1 * h[:, :DIM], dis2 * h[:, DIM:],
         jnp.zeros((bl, GW - 2 * DIM), jnp.float32)], axis=1)


def _tc_prep_body(degp_ref, xp_ref, wc_ref, g_ref, hs_ref, dis_ref):
    deg1 = degp_ref[0, 0] + degp_ref[0, 1] + 1.0
    deg2 = degp_ref[1, 0] + degp_ref[1, 1] + 1.0
    dis1 = 1.0 / jnp.sqrt(deg1)
    dis2 = 1.0 / jnp.sqrt(deg2)
    h = jnp.dot(xp_ref[...], wc_ref[...], preferred_element_type=jnp.float32)
    g_ref[...] = _pack_g(dis1, dis2, h)
    hs_ref[...] = h
    dis_ref[...] = jnp.concatenate([dis1, dis2], axis=1)


def _tc_layer_body(p_ref, hs_ref, dis_ref, b1_ref, b2_ref, wc_ref,
                   g_ref, hs2_ref):
    dis1 = dis_ref[:, 0:1]
    dis2 = dis_ref[:, 1:2]
    agg1 = p_ref[0, 0][:, :DIM] + p_ref[0, 1][:, :DIM]
    agg2 = (p_ref[1, 0][:, DIM:2 * DIM] + p_ref[1, 1][:, DIM:2 * DIM]
            - p_ref[0, 0][:, DIM:2 * DIM] - p_ref[0, 1][:, DIM:2 * DIM])
    x1 = jnp.maximum(dis1 * agg1 + dis1 * dis1 * hs_ref[:, :DIM] + b1_ref[...], 0.0)
    x2 = jnp.maximum(dis2 * agg2 + dis2 * dis2 * hs_ref[:, DIM:] + b2_ref[...], 0.0)
    x12 = jnp.concatenate([x1, x2], axis=1)
    h2 = jnp.dot(x12, wc_ref[...], preferred_element_type=jnp.float32)
    g_ref[...] = _pack_g(dis1, dis2, h2)
    hs2_ref[...] = h2


def _tc_pre_body(q_ref, hs_ref, dis_ref, b1_ref, b2_ref, wih_ref, bb_ref,
                 pre_ref):
    dis1 = dis_ref[:, 0:1]
    dis2 = dis_ref[:, 1:2]
    agg1 = q_ref[0, 0][:, :DIM] + q_ref[0, 1][:, :DIM]
    agg2 = (q_ref[1, 0][:, DIM:2 * DIM] + q_ref[1, 1][:, DIM:2 * DIM]
            - q_ref[0, 0][:, DIM:2 * DIM] - q_ref[0, 1][:, DIM:2 * DIM])
    x1 = jnp.maximum(dis1 * agg1 + dis1 * dis1 * hs_ref[:, :DIM] + b1_ref[...], 0.0)
    x2 = jnp.maximum(dis2 * agg2 + dis2 * dis2 * hs_ref[:, DIM:] + b2_ref[...], 0.0)
    x12 = jnp.concatenate([x1, x2], axis=1)
    pre_ref[...] = (
        jnp.dot(x12, wih_ref[...], preferred_element_type=jnp.float32)
        + bb_ref[...]
    )


_UNROLL = 16
LW = 4 * GW


def _tc_lstm_body(pre_ref, whh_ref, ys_ref, hn_ref, cn_ref):
    whh = whh_ref[...]

    def blk(tb, carry):
        h, c = carry
        pre8 = pre_ref[pl.ds(tb * _UNROLL, _UNROLL), :]
        outs = []
        for k in range(_UNROLL):
            g = pre8[k:k + 1, :] + jnp.dot(
                h[:, :DIM], whh, preferred_element_type=jnp.float32)
            si = 1.0 / (1.0 + jnp.exp(-g[:, 0:GW]))
            sf = 1.0 / (1.0 + jnp.exp(-g[:, GW:2 * GW]))
            sg = jnp.tanh(g[:, 2 * GW:3 * GW])
            so = 1.0 / (1.0 + jnp.exp(-g[:, 3 * GW:]))
            c = sf * c + si * sg
            h = so * jnp.tanh(c)
            outs.append(h[:, :DIM])
        ys_ref[pl.ds(tb * _UNROLL, _UNROLL), :] = jnp.concatenate(outs, axis=0)
        return (h, c)

    z = jnp.zeros((1, GW), jnp.float32)
    h, c = lax.fori_loop(0, N // _UNROLL, blk, (z, z))
    hn_ref[...] = h[:, :DIM]
    cn_ref[...] = c[:, :DIM]


_GRID = NPAD // BL


def _prep_call(degp4, xp, wc):
    return pl.pallas_call(
        _tc_prep_body,
        grid=(_GRID,),
        in_specs=[
            pl.BlockSpec((2, NC, BL, 1), lambda i: (0, 0, i, 0)),
            pl.BlockSpec((BL, F_IN), lambda i: (i, 0)),
            pl.BlockSpec((F_IN, 2 * DIM), lambda i: (0, 0)),
        ],
        out_specs=[
            pl.BlockSpec((BL, GW), lambda i: (i, 0)),
            pl.BlockSpec((BL, 2 * DIM), lambda i: (i, 0)),
            pl.BlockSpec((BL, 2), lambda i: (i, 0)),
        ],
        out_shape=[
            jax.ShapeDtypeStruct((NPAD, GW), jnp.float32),
            jax.ShapeDtypeStruct((NPAD, 2 * DIM), jnp.float32),
            jax.ShapeDtypeStruct((NPAD, 2), jnp.float32),
        ],
    )(degp4, xp, wc)


def _layer_call(p, hs, dis, b1, b2, wc):
    return pl.pallas_call(
        _tc_layer_body,
        grid=(_GRID,),
        in_specs=[
            pl.BlockSpec((2, NC, BL, GW), lambda i: (0, 0, i, 0)),
            pl.BlockSpec((BL, 2 * DIM), lambda i: (i, 0)),
            pl.BlockSpec((BL, 2), lambda i: (i, 0)),
            pl.BlockSpec((1, DIM), lambda i: (0, 0)),
            pl.BlockSpec((1, DIM), lambda i: (0, 0)),
            pl.BlockSpec((2 * DIM, 2 * DIM), lambda i: (0, 0)),
        ],
        out_specs=[
            pl.BlockSpec((BL, GW), lambda i: (i, 0)),
            pl.BlockSpec((BL, 2 * DIM), lambda i: (i, 0)),
        ],
        out_shape=[
            jax.ShapeDtypeStruct((NPAD, GW), jnp.float32),
            jax.ShapeDtypeStruct((NPAD, 2 * DIM), jnp.float32),
        ],
    )(p, hs, dis, b1, b2, wc)


def _pre_call(q, hs2, dis, b1, b2, wih_big, bb):
    return pl.pallas_call(
        _tc_pre_body,
        grid=(_GRID,),
        in_specs=[
            pl.BlockSpec((2, NC, BL, GW), lambda i: (0, 0, i, 0)),
            pl.BlockSpec((BL, 2 * DIM), lambda i: (i, 0)),
            pl.BlockSpec((BL, 2), lambda i: (i, 0)),
            pl.BlockSpec((1, DIM), lambda i: (0, 0)),
            pl.BlockSpec((1, DIM), lambda i: (0, 0)),
            pl.BlockSpec((2 * DIM, LW), lambda i: (0, 0)),
            pl.BlockSpec((1, LW), lambda i: (0, 0)),
        ],
        out_specs=[pl.BlockSpec((BL, LW), lambda i: (i, 0))],
        out_shape=[jax.ShapeDtypeStruct((NPAD, LW), jnp.float32)],
    )(q, hs2, dis, b1, b2, wih_big, bb)[0]


def _lstm_call(pre, whh_big):
    return pl.pallas_call(
        _tc_lstm_body,
        out_shape=[
            jax.ShapeDtypeStruct((N, DIM), jnp.float32),
            jax.ShapeDtypeStruct((1, DIM), jnp.float32),
            jax.ShapeDtypeStruct((1, DIM), jnp.float32),
        ],
    )(pre, whh_big)


def _tile_gates(w):
    return jnp.concatenate(
        [jnp.tile(w[:, g * DIM:(g + 1) * DIM], (1, 4)) for g in range(4)],
        axis=1)


def _pad_edges(ei):
    src, dst = ei[0], ei[1]
    pad = EPAD - E
    fill = N + (jnp.arange(pad, dtype=jnp.int32) % (NPAD - N))
    srcp = jnp.concatenate([src, fill]).reshape(NC, NS, K, CH)
    dstp = jnp.concatenate([dst, fill]).reshape(NC, NS, K, CH)
    return srcp, dstp


def kernel(x, edge_index, edge_index2, W11, b11, W12, b12, W21, b21, W22, b22,
           W_ih, W_hh, b_ih, b_hh):
    xp = jnp.pad(x, ((0, NPAD - N), (0, 0)))
    s1, d1 = _pad_edges(edge_index)
    s2, d2 = _pad_edges(edge_index2)

    degp = _deg_kernel_fn()(d1, d2)
    degp4 = degp.reshape(2, NC, NPAD, 1)
    wc1 = jnp.concatenate([W11, W12], axis=1)
    g, hs, dis = _prep_call(degp4, xp, wc1)

    p = _msg_kernel_fn()(g, s1, d1, s2, d2)
    wc2 = jnp.concatenate([W21, W22], axis=1)
    g2nd, hs2 = _layer_call(p, hs, dis, b11.reshape(1, DIM),
                            b12.reshape(1, DIM), wc2)

    q = _msg_kernel_fn()(g2nd, s1, d1, s2, d2)
    wih_big = _tile_gates(W_ih.T)
    whh_big = _tile_gates(W_hh.T)
    bb = _tile_gates((b_ih + b_hh).reshape(1, 4 * DIM))

    pre = _pre_call(q, hs2, dis, b21.reshape(1, DIM), b22.reshape(1, DIM),
                    wih_big, bb)
    ys, hn, cn = _lstm_call(pre, whh_big)
    return ys[None], hn[None], cn[None]


# speedup vs baseline: 18.8826x; 1.0326x over previous
import functools

import jax
import jax.numpy as jnp
from jax import lax
from jax.experimental import pallas as pl
from jax.experimental.pallas import tpu as pltpu
from jax.experimental.pallas import tpu_sc as plsc

N = 10000
F_IN = 128
DIM = 32
E = 320000

NPAD = 10240
NC, NS = 2, 16
NW = NC * NS
CH = 128
K = 79
EPT = K * CH
EPAD = NW * EPT
RPT = NPAD // NS
BL = 2048


def _mesh():
    return plsc.VectorSubcoreMesh(
        core_axis_name="c", subcore_axis_name="s", num_cores=NC, num_subcores=NS
    )


def _sc_deg_body(d1_ref, d2_ref, out_ref, idx_v, ones_v, zer_v, acc1, acc2):
    cid = lax.axis_index("c")
    sid = lax.axis_index("s")
    for i in range(CH // 16):
        ones_v[pl.ds(i * 16, 16)] = jnp.full((16,), 1.0, jnp.float32)
    for i in range(RPT // 16):
        zer_v[pl.ds(i * 16, 16)] = jnp.zeros((16,), jnp.float32)
    sl = pl.ds(sid * RPT, RPT)
    pltpu.sync_copy(zer_v, acc1.at[sl])
    pltpu.sync_copy(zer_v, acc2.at[sl])
    plsc.subcore_barrier()
    for d_ref, acc in ((d1_ref, acc1), (d2_ref, acc2)):
        pltpu.sync_copy(d_ref.at[cid, sid], idx_v)

        def body(j, carry, acc=acc):
            pltpu.sync_copy(ones_v, acc.at[idx_v.at[j]], add=True)
            return carry

        lax.fori_loop(0, K, body, 0)
    plsc.subcore_barrier()
    pltpu.sync_copy(acc1.at[sl], out_ref.at[0, cid, sl])
    pltpu.sync_copy(acc2.at[sl], out_ref.at[1, cid, sl])


@functools.cache
def _deg_kernel_fn():
    return pl.kernel(
        _sc_deg_body,
        out_type=jax.ShapeDtypeStruct((2, NC, NPAD), jnp.float32),
        mesh=_mesh(),
        scratch_types=[
            pltpu.VMEM((K, CH), jnp.int32),
            pltpu.VMEM((CH,), jnp.float32),
            pltpu.VMEM((RPT,), jnp.float32),
            pltpu.VMEM_SHARED((NPAD,), jnp.float32),
            pltpu.VMEM_SHARED((NPAD,), jnp.float32),
        ],
    )


GW = 128


def _sc_msg_body(g_ref, s1_ref, d1_ref, s2_ref, d2_ref, out_ref,
                 sidx, didx, rows, acc, sem):
    cid = lax.axis_index("c")
    sid = lax.axis_index("s")
    z16 = jnp.zeros((16,), jnp.float32)

    def zbody(i, carry):
        for u in range(GW // 16):
            rows[i, pl.ds(u * 16, 16)] = z16
        return carry

    lax.fori_loop(0, CH, zbody, 0)
    for r in range(RPT // CH):
        pltpu.sync_copy(rows, acc.at[pl.ds(sid * RPT + r * CH, CH)])
    plsc.subcore_barrier()
    sl = pl.ds(sid * RPT, RPT)
    for set_i, (s_ref, d_ref) in enumerate(((s1_ref, d1_ref), (s2_ref, d2_ref))):
        pltpu.sync_copy(s_ref.at[cid, sid], sidx)
        pltpu.sync_copy(d_ref.at[cid, sid], didx)

        def body(j, carry):
            pltpu.async_copy(g_ref.at[sidx.at[j]], rows, sem).wait()
            pltpu.sync_copy(rows, acc.at[didx.at[j]], add=True)
            return carry

        lax.fori_loop(0, K, body, 0)
        plsc.subcore_barrier()
        pltpu.sync_copy(acc.at[sl], out_ref.at[set_i, cid, sl])
        plsc.subcore_barrier()


@functools.cache
def _msg_kernel_fn():
    return pl.kernel(
        _sc_msg_body,
        out_type=jax.ShapeDtypeStruct((2, NC, NPAD, GW), jnp.float32),
        mesh=_mesh(),
        scratch_types=[
            pltpu.VMEM((K, CH), jnp.int32),
            pltpu.VMEM((K, CH), jnp.int32),
            pltpu.VMEM((CH, GW), jnp.float32),
            pltpu.VMEM_SHARED((NPAD, GW), jnp.float32),
            pltpu.SemaphoreType.DMA,
        ],
    )


def _pack_g(dis1, dis2, h):
    bl = h.shape[0]
    return jnp.concatenate(
        [dis1 * h[:, :DIM], dis2 * h[:, DIM:],
         jnp.zeros((bl, GW - 2 * DIM), jnp.float32)], axis=1)


def _tc_prep_body(degp_ref, xp_ref, wc_ref, g_ref, hs_ref, dis_ref):
    deg1 = degp_ref[0, 0] + degp_ref[0, 1] + 1.0
    deg2 = degp_ref[1, 0] + degp_ref[1, 1] + 1.0
    dis1 = 1.0 / jnp.sqrt(deg1)
    dis2 = 1.0 / jnp.sqrt(deg2)
    h = jnp.dot(xp_ref[...], wc_ref[...], preferred_element_type=jnp.float32)
    g_ref[...] = _pack_g(dis1, dis2, h)
    hs_ref[...] = h
    dis_ref[...] = jnp.concatenate([dis1, dis2], axis=1)


def _tc_layer_body(p_ref, hs_ref, dis_ref, b1_ref, b2_ref, wc_ref,
                   g_ref, hs2_ref):
    dis1 = dis_ref[:, 0:1]
    dis2 = dis_ref[:, 1:2]
    agg1 = p_ref[0, 0][:, :DIM] + p_ref[0, 1][:, :DIM]
    agg2 = (p_ref[1, 0][:, DIM:2 * DIM] + p_ref[1, 1][:, DIM:2 * DIM]
            - p_ref[0, 0][:, DIM:2 * DIM] - p_ref[0, 1][:, DIM:2 * DIM])
    x1 = jnp.maximum(dis1 * agg1 + dis1 * dis1 * hs_ref[:, :DIM] + b1_ref[...], 0.0)
    x2 = jnp.maximum(dis2 * agg2 + dis2 * dis2 * hs_ref[:, DIM:] + b2_ref[...], 0.0)
    x12 = jnp.concatenate([x1, x2], axis=1)
    h2 = jnp.dot(x12, wc_ref[...], preferred_element_type=jnp.float32)
    g_ref[...] = _pack_g(dis1, dis2, h2)
    hs2_ref[...] = h2


def _tc_pre_body(q_ref, hs_ref, dis_ref, b1_ref, b2_ref, wih_ref, bb_ref,
                 pre_ref):
    dis1 = dis_ref[:, 0:1]
    dis2 = dis_ref[:, 1:2]
    agg1 = q_ref[0, 0][:, :DIM] + q_ref[0, 1][:, :DIM]
    agg2 = (q_ref[1, 0][:, DIM:2 * DIM] + q_ref[1, 1][:, DIM:2 * DIM]
            - q_ref[0, 0][:, DIM:2 * DIM] - q_ref[0, 1][:, DIM:2 * DIM])
    x1 = jnp.maximum(dis1 * agg1 + dis1 * dis1 * hs_ref[:, :DIM] + b1_ref[...], 0.0)
    x2 = jnp.maximum(dis2 * agg2 + dis2 * dis2 * hs_ref[:, DIM:] + b2_ref[...], 0.0)
    x12 = jnp.concatenate([x1, x2], axis=1)
    pre_ref[...] = (
        jnp.dot(x12, wih_ref[...], preferred_element_type=jnp.float32)
        + bb_ref[...]
    )


_UNROLL = 16
LW = 4 * GW


def _tc_lstm_body(pre_ref, whh_ref, ys_ref, hn_ref, cn_ref):
    whh = whh_ref[...]

    def blk(tb, carry):
        h, c = carry
        pre8 = pre_ref[pl.ds(tb * _UNROLL, _UNROLL), :]
        outs = []
        for k in range(_UNROLL):
            g = pre8[k:k + 1, :] + jnp.dot(
                h[:, :DIM], whh, preferred_element_type=jnp.float32)
            si = 0.5 + 0.5 * jnp.tanh(g[:, 0:GW])
            sf = 0.5 + 0.5 * jnp.tanh(g[:, GW:2 * GW])
            sg = jnp.tanh(g[:, 2 * GW:3 * GW])
            so = 0.5 + 0.5 * jnp.tanh(g[:, 3 * GW:])
            c = sf * c + si * sg
            h = so * jnp.tanh(c)
            outs.append(h[:, :DIM])
        ys_ref[pl.ds(tb * _UNROLL, _UNROLL), :] = jnp.concatenate(outs, axis=0)
        return (h, c)

    z = jnp.zeros((1, GW), jnp.float32)
    h, c = lax.fori_loop(0, N // _UNROLL, blk, (z, z))
    hn_ref[...] = h[:, :DIM]
    cn_ref[...] = c[:, :DIM]


_GRID = NPAD // BL


def _prep_call(degp4, xp, wc):
    return pl.pallas_call(
        _tc_prep_body,
        grid=(_GRID,),
        in_specs=[
            pl.BlockSpec((2, NC, BL, 1), lambda i: (0, 0, i, 0)),
            pl.BlockSpec((BL, F_IN), lambda i: (i, 0)),
            pl.BlockSpec((F_IN, 2 * DIM), lambda i: (0, 0)),
        ],
        out_specs=[
            pl.BlockSpec((BL, GW), lambda i: (i, 0)),
            pl.BlockSpec((BL, 2 * DIM), lambda i: (i, 0)),
            pl.BlockSpec((BL, 2), lambda i: (i, 0)),
        ],
        out_shape=[
            jax.ShapeDtypeStruct((NPAD, GW), jnp.float32),
            jax.ShapeDtypeStruct((NPAD, 2 * DIM), jnp.float32),
            jax.ShapeDtypeStruct((NPAD, 2), jnp.float32),
        ],
    )(degp4, xp, wc)


def _layer_call(p, hs, dis, b1, b2, wc):
    return pl.pallas_call(
        _tc_layer_body,
        grid=(_GRID,),
        in_specs=[
            pl.BlockSpec((2, NC, BL, GW), lambda i: (0, 0, i, 0)),
            pl.BlockSpec((BL, 2 * DIM), lambda i: (i, 0)),
            pl.BlockSpec((BL, 2), lambda i: (i, 0)),
            pl.BlockSpec((1, DIM), lambda i: (0, 0)),
            pl.BlockSpec((1, DIM), lambda i: (0, 0)),
            pl.BlockSpec((2 * DIM, 2 * DIM), lambda i: (0, 0)),
        ],
        out_specs=[
            pl.BlockSpec((BL, GW), lambda i: (i, 0)),
            pl.BlockSpec((BL, 2 * DIM), lambda i: (i, 0)),
        ],
        out_shape=[
            jax.ShapeDtypeStruct((NPAD, GW), jnp.float32),
            jax.ShapeDtypeStruct((NPAD, 2 * DIM), jnp.float32),
        ],
    )(p, hs, dis, b1, b2, wc)


def _pre_call(q, hs2, dis, b1, b2, wih_big, bb):
    return pl.pallas_call(
        _tc_pre_body,
        grid=(_GRID,),
        in_specs=[
            pl.BlockSpec((2, NC, BL, GW), lambda i: (0, 0, i, 0)),
            pl.BlockSpec((BL, 2 * DIM), lambda i: (i, 0)),
            pl.BlockSpec((BL, 2), lambda i: (i, 0)),
            pl.BlockSpec((1, DIM), lambda i: (0, 0)),
            pl.BlockSpec((1, DIM), lambda i: (0, 0)),
            pl.BlockSpec((2 * DIM, LW), lambda i: (0, 0)),
            pl.BlockSpec((1, LW), lambda i: (0, 0)),
        ],
        out_specs=[pl.BlockSpec((BL, LW), lambda i: (i, 0))],
        out_shape=[jax.ShapeDtypeStruct((NPAD, LW), jnp.float32)],
    )(q, hs2, dis, b1, b2, wih_big, bb)[0]


def _lstm_call(pre, whh_big):
    return pl.pallas_call(
        _tc_lstm_body,
        out_shape=[
            jax.ShapeDtypeStruct((N, DIM), jnp.float32),
            jax.ShapeDtypeStruct((1, DIM), jnp.float32),
            jax.ShapeDtypeStruct((1, DIM), jnp.float32),
        ],
    )(pre, whh_big)


def _tile_gates(w):
    return jnp.concatenate(
        [jnp.tile(w[:, g * DIM:(g + 1) * DIM], (1, 4)) for g in range(4)],
        axis=1)


def _pad_edges(ei):
    src, dst = ei[0], ei[1]
    pad = EPAD - E
    fill = N + (jnp.arange(pad, dtype=jnp.int32) % (NPAD - N))
    srcp = jnp.concatenate([src, fill]).reshape(NC, NS, K, CH)
    dstp = jnp.concatenate([dst, fill]).reshape(NC, NS, K, CH)
    return srcp, dstp


def kernel(x, edge_index, edge_index2, W11, b11, W12, b12, W21, b21, W22, b22,
           W_ih, W_hh, b_ih, b_hh):
    xp = jnp.pad(x, ((0, NPAD - N), (0, 0)))
    s1, d1 = _pad_edges(edge_index)
    s2, d2 = _pad_edges(edge_index2)

    degp = _deg_kernel_fn()(d1, d2)
    degp4 = degp.reshape(2, NC, NPAD, 1)
    wc1 = jnp.concatenate([W11, W12], axis=1)
    g, hs, dis = _prep_call(degp4, xp, wc1)

    p = _msg_kernel_fn()(g, s1, d1, s2, d2)
    wc2 = jnp.concatenate([W21, W22], axis=1)
    g2nd, hs2 = _layer_call(p, hs, dis, b11.reshape(1, DIM),
                            b12.reshape(1, DIM), wc2)

    q = _msg_kernel_fn()(g2nd, s1, d1, s2, d2)
    gate_scale = jnp.concatenate(
        [jnp.full((1, GW), 0.5, jnp.float32), jnp.full((1, GW), 0.5, jnp.float32),
         jnp.ones((1, GW), jnp.float32), jnp.full((1, GW), 0.5, jnp.float32)],
        axis=1)
    wih_big = _tile_gates(W_ih.T) * gate_scale
    whh_big = _tile_gates(W_hh.T) * gate_scale
    bb = _tile_gates((b_ih + b_hh).reshape(1, 4 * DIM)) * gate_scale

    pre = _pre_call(q, hs2, dis, b21.reshape(1, DIM), b22.reshape(1, DIM),
                    wih_big, bb)
    ys, hn, cn = _lstm_call(pre, whh_big)
    return ys[None], hn[None], cn[None]
